# baseline (speedup 1.0000x reference)
import dataclasses
import functools

import jax
import jax.numpy as jnp
from jax import lax
from jax.experimental import pallas as pl
from jax.experimental.pallas import tpu as pltpu
from jax.experimental.pallas import tpu_sc as plsc

_N = 10000
_E = 320000
_DIN = 128
_DH = 32
_DOUT = 64

_NC = 2
_NS = 16
_NW = _NC * _NS
_CH = 128
_NCHUNK = 80
_EP = _NW * _NCHUNK * _CH
_NP = 10240
_RPS = _NP // _NS
_ZB = 128

_BR = 1280
_NB = 8


_NQUAD = _NCHUNK // 4


def _seg_sum_sc(table, src2, dst2, ew2):
    mesh = plsc.VectorSubcoreMesh(core_axis_name="c", subcore_axis_name="s")
    cp = pltpu.CompilerParams()
    if "needs_layout_passes" in pltpu.CompilerParams.__dataclass_fields__:
        cp = dataclasses.replace(cp, needs_layout_passes=False)
    if "use_tc_tiling_on_sc" in pltpu.CompilerParams.__dataclass_fields__:
        cp = dataclasses.replace(cp, use_tc_tiling_on_sc=False)

    @functools.partial(
        pl.kernel,
        compiler_params=cp,
        out_type=jax.ShapeDtypeStruct((_NC * _NP, _DH), jnp.float32),
        mesh=mesh,
        scratch_types=[
            pltpu.VMEM((_NCHUNK, _CH), jnp.int32),
            pltpu.VMEM((_NCHUNK, _CH), jnp.int32),
            pltpu.VMEM((_NCHUNK, _CH), jnp.float32),
            pltpu.VMEM((_CH, _DH), jnp.float32),
            pltpu.VMEM((_CH, _DH), jnp.float32),
            pltpu.VMEM((_CH, _DH), jnp.float32),
            pltpu.VMEM((_CH, _DH), jnp.float32),
            pltpu.VMEM((_ZB, _DH), jnp.float32),
            pltpu.VMEM_SHARED((_NP, _DH), jnp.float32),
            pltpu.SemaphoreType.DMA,
            pltpu.SemaphoreType.DMA,
            pltpu.SemaphoreType.DMA,
            pltpu.SemaphoreType.DMA,
            pltpu.SemaphoreType.DMA,
            pltpu.SemaphoreType.DMA,
            pltpu.SemaphoreType.DMA,
            pltpu.SemaphoreType.DMA,
            pltpu.SemaphoreType.DMA,
        ],
    )
    def k(table_hbm, src2_hbm, dst2_hbm, ew2_hbm, out_hbm,
          sidx2, didx2, wv2, rows_a, rows_b, rows_c, rows_d, zbuf, acc,
          gsem_a, gsem_b, gsem_c, gsem_d,
          ssem_a, ssem_b, ssem_c, ssem_d, isem):
        cid = lax.axis_index("c")
        sid = lax.axis_index("s")
        wid = sid * _NC + cid
        crow = wid * _NCHUNK

        ld_s = pltpu.async_copy(src2_hbm.at[pl.ds(crow, _NCHUNK)], sidx2, isem)
        ld_d = pltpu.async_copy(dst2_hbm.at[pl.ds(crow, _NCHUNK)], didx2, isem)
        ld_w = pltpu.async_copy(ew2_hbm.at[pl.ds(crow, _NCHUNK)], wv2, isem)

        zero16 = jnp.zeros((16,), jnp.float32)

        @pl.loop(0, _ZB)
        def _zfill(i):
            zbuf[i, pl.ds(0, 16)] = zero16
            zbuf[i, pl.ds(16, 16)] = zero16

        @pl.loop(0, 5)
        def _zcopy(j):
            pltpu.sync_copy(zbuf, acc.at[pl.ds(sid * _RPS + j * _ZB, _ZB)])

        ld_s.wait()
        ld_d.wait()
        ld_w.wait()
        plsc.subcore_barrier()

        def gather(c, rows, sem):
            pltpu.async_copy(table_hbm.at[sidx2.at[c]], rows, sem)

        def wait_gather(c, rows, sem):
            pltpu.make_async_copy(table_hbm.at[sidx2.at[c]], rows, sem).wait()

        def mult(rows, c):
            @plsc.parallel_loop(0, _CH, step=16)
            def _grp(g):
                wgrp = wv2[c, pl.ds(g, 16)]
                for u in range(16):
                    w = wgrp[u]
                    rows[g + u, pl.ds(0, 16)] = rows[g + u, pl.ds(0, 16)] * w
                    rows[g + u, pl.ds(16, 16)] = rows[g + u, pl.ds(16, 16)] * w

        def scatter(c, rows, sem):
            pltpu.async_copy(rows, acc.at[didx2.at[c]], sem, add=True)

        def wait_scatter(c, rows, sem):
            pltpu.make_async_copy(rows, acc.at[didx2.at[c]], sem).wait()

        bufs = [(rows_a, gsem_a, ssem_a), (rows_b, gsem_b, ssem_b),
                (rows_c, gsem_c, ssem_c), (rows_d, gsem_d, ssem_d)]

        gather(0, rows_a, gsem_a)
        gather(1, rows_b, gsem_b)
        gather(2, rows_c, gsem_c)

        @pl.loop(0, _NQUAD)
        def _quad(t):
            c = 4 * t
            for j in range(4):
                bx, gx, sx = bufs[j]
                pv_rows, pv_gsem, pv_ssem = bufs[(j + 3) % 4]
                wait_gather(c + j, bx, gx)
                mult(bx, c + j)
                scatter(c + j, bx, sx)
                if j == 0:
                    @pl.when(t > 0)
                    def _(pv_rows=pv_rows, pv_ssem=pv_ssem, cw=c - 1):
                        wait_scatter(cw, pv_rows, pv_ssem)
                    gather(c + 3, pv_rows, pv_gsem)
                else:
                    wait_scatter(c + j - 1, pv_rows, pv_ssem)
                    nxt = c + j + 3

                    @pl.when(t < _NQUAD - 1)
                    def _(pv_rows=pv_rows, pv_gsem=pv_gsem, nxt=nxt):
                        gather(nxt, pv_rows, pv_gsem)

        wait_scatter(_NCHUNK - 1, rows_d, ssem_d)

        plsc.subcore_barrier()

        @pl.loop(0, 5)
        def _wb(j):
            r0 = sid * _RPS + j * _ZB
            pltpu.sync_copy(acc.at[pl.ds(r0, _ZB)],
                            out_hbm.at[pl.ds(cid * _NP + r0, _ZB)])

    return k(table, src2, dst2, ew2)


def _tc_proj0(xp, wr, wo, b):
    def body(x_ref, wr_ref, wo_ref, b_ref, t_ref, r_ref):
        xb = x_ref[...]
        t_ref[...] = jnp.dot(xb, wr_ref[...], preferred_element_type=jnp.float32)
        r_ref[...] = jnp.dot(xb, wo_ref[...], preferred_element_type=jnp.float32) + b_ref[...]

    return pl.pallas_call(
        body,
        grid=(_NB,),
        in_specs=[
            pl.BlockSpec((_BR, _DIN), lambda i: (i, 0)),
            pl.BlockSpec((_DIN, _DH), lambda i: (0, 0)),
            pl.BlockSpec((_DIN, _DH), lambda i: (0, 0)),
            pl.BlockSpec((1, _DH), lambda i: (0, 0)),
        ],
        out_specs=[
            pl.BlockSpec((_BR, _DH), lambda i: (i, 0)),
            pl.BlockSpec((_BR, _DH), lambda i: (i, 0)),
        ],
        out_shape=[jax.ShapeDtypeStruct((_NP, _DH), jnp.float32)] * 2,
    )(xp, wr, wo, b.reshape(1, _DH))


def _tc_mid(ap, r_prev, wr, wo, b):
    def body(a0_ref, a1_ref, rp_ref, wr_ref, wo_ref, b_ref, t_ref, r_ref):
        h = a0_ref[...] + a1_ref[...] + rp_ref[...]
        h = jnp.where(h > 0, h, 0.01 * h)
        t_ref[...] = jnp.dot(h, wr_ref[...], preferred_element_type=jnp.float32)
        r_ref[...] = jnp.dot(h, wo_ref[...], preferred_element_type=jnp.float32) + b_ref[...]

    return pl.pallas_call(
        body,
        grid=(_NB,),
        in_specs=[
            pl.BlockSpec((_BR, _DH), lambda i: (i, 0)),
            pl.BlockSpec((_BR, _DH), lambda i: (i + _NB, 0)),
            pl.BlockSpec((_BR, _DH), lambda i: (i, 0)),
            pl.BlockSpec((_DH, _DH), lambda i: (0, 0)),
            pl.BlockSpec((_DH, _DH), lambda i: (0, 0)),
            pl.BlockSpec((1, _DH), lambda i: (0, 0)),
        ],
        out_specs=[
            pl.BlockSpec((_BR, _DH), lambda i: (i, 0)),
            pl.BlockSpec((_BR, _DH), lambda i: (i, 0)),
        ],
        out_shape=[jax.ShapeDtypeStruct((_NP, _DH), jnp.float32)] * 2,
    )(ap, ap, r_prev, wr, wo, b.reshape(1, _DH))


def _tc_last_pre(ap, r_prev, wo, b):
    def body(a0_ref, a1_ref, rp_ref, wo_ref, b_ref, h_ref, r_ref):
        h = a0_ref[...] + a1_ref[...] + rp_ref[...]
        h = jnp.where(h > 0, h, 0.01 * h)
        h_ref[...] = h
        r_ref[...] = jnp.dot(h, wo_ref[...], preferred_element_type=jnp.float32) + b_ref[...]

    return pl.pallas_call(
        body,
        grid=(_NB,),
        in_specs=[
            pl.BlockSpec((_BR, _DH), lambda i: (i, 0)),
            pl.BlockSpec((_BR, _DH), lambda i: (i + _NB, 0)),
            pl.BlockSpec((_BR, _DH), lambda i: (i, 0)),
            pl.BlockSpec((_DH, _DOUT), lambda i: (0, 0)),
            pl.BlockSpec((1, _DOUT), lambda i: (0, 0)),
        ],
        out_specs=[
            pl.BlockSpec((_BR, _DH), lambda i: (i, 0)),
            pl.BlockSpec((_BR, _DOUT), lambda i: (i, 0)),
        ],
        out_shape=[
            jax.ShapeDtypeStruct((_NP, _DH), jnp.float32),
            jax.ShapeDtypeStruct((_NP, _DOUT), jnp.float32),
        ],
    )(ap, ap, r_prev, wo, b.reshape(1, _DOUT))


def _tc_final(ap, r2, wr):
    def body(a0_ref, a1_ref, r2_ref, wr_ref, o_ref):
        a = a0_ref[...] + a1_ref[...]
        o_ref[...] = jnp.dot(a, wr_ref[...], preferred_element_type=jnp.float32) + r2_ref[...]

    return pl.pallas_call(
        body,
        grid=(_NB,),
        in_specs=[
            pl.BlockSpec((_BR, _DH), lambda i: (i, 0)),
            pl.BlockSpec((_BR, _DH), lambda i: (i + _NB, 0)),
            pl.BlockSpec((_BR, _DOUT), lambda i: (i, 0)),
            pl.BlockSpec((_DH, _DOUT), lambda i: (0, 0)),
        ],
        out_specs=pl.BlockSpec((_BR, _DOUT), lambda i: (i, 0)),
        out_shape=jax.ShapeDtypeStruct((_NP, _DOUT), jnp.float32),
    )(ap, ap, r2, wr)


def kernel(x, edge_index, edge_weights,
           W_rel0, W_root0, b0,
           W_rel1, W_root1, b1,
           W_rel2, W_root2, b2):
    pad_e = _EP - _E
    src2 = jnp.pad(edge_index[0].astype(jnp.int32), (0, pad_e)).reshape(_EP // _CH, _CH)
    dst2 = jnp.pad(edge_index[1].astype(jnp.int32), (0, pad_e)).reshape(_EP // _CH, _CH)
    ew2 = jnp.pad(edge_weights.astype(jnp.float32), (0, pad_e)).reshape(_EP // _CH, _CH)
    xp = jnp.pad(x, ((0, _NP - _N), (0, 0)))

    def seg(table):
        return _seg_sum_sc(table, src2, dst2, ew2)

    t0, r0 = _tc_proj0(xp, W_rel0, W_root0, b0)
    a0 = seg(t0)
    t1, r1 = _tc_mid(a0, r0, W_rel1, W_root1, b1)
    a1 = seg(t1)
    h2, r2 = _tc_last_pre(a1, r1, W_root2, b2)
    a2 = seg(h2)
    return _tc_final(a2, r2, W_rel2)[:_N]


# speedup vs baseline: 2.1431x; 2.1431x over previous
import dataclasses
import functools

import jax
import jax.numpy as jnp
from jax import lax
from jax.experimental import pallas as pl
from jax.experimental.pallas import tpu as pltpu
from jax.experimental.pallas import tpu_sc as plsc

_N = 10000
_E = 320000
_DIN = 128
_DH = 32
_DOUT = 64

_NC = 2
_NS = 16
_NW = _NC * _NS
_CH = 128
_NCHUNK = 80
_EP = _NW * _NCHUNK * _CH
_NP = 10240
_RPS = _NP // _NS
_ZB = 128

_BR = 1280
_NB = 8


_NQUAD = _NCHUNK // 4


def _seg_sum_sc(table, src2, dst2, ew2):
    mesh = plsc.VectorSubcoreMesh(core_axis_name="c", subcore_axis_name="s")
    cp = pltpu.CompilerParams()
    if "needs_layout_passes" in pltpu.CompilerParams.__dataclass_fields__:
        cp = dataclasses.replace(cp, needs_layout_passes=False)
    if "use_tc_tiling_on_sc" in pltpu.CompilerParams.__dataclass_fields__:
        cp = dataclasses.replace(cp, use_tc_tiling_on_sc=False)

    @functools.partial(
        pl.kernel,
        compiler_params=cp,
        out_type=jax.ShapeDtypeStruct((_NC * _NP, _DH), jnp.float32),
        mesh=mesh,
        scratch_types=[
            pltpu.VMEM((_NCHUNK, _CH), jnp.int32),
            pltpu.VMEM((_NCHUNK, _CH), jnp.int32),
            pltpu.VMEM((_NCHUNK, _CH), jnp.float32),
            pltpu.VMEM((_CH, _DH), jnp.float32),
            pltpu.VMEM((_CH, _DH), jnp.float32),
            pltpu.VMEM((_CH, _DH), jnp.float32),
            pltpu.VMEM((_CH, _DH), jnp.float32),
            pltpu.VMEM((_ZB, _DH), jnp.float32),
            pltpu.VMEM_SHARED((_NP, _DH), jnp.float32),
            pltpu.SemaphoreType.DMA,
            pltpu.SemaphoreType.DMA,
            pltpu.SemaphoreType.DMA,
            pltpu.SemaphoreType.DMA,
            pltpu.SemaphoreType.DMA,
            pltpu.SemaphoreType.DMA,
            pltpu.SemaphoreType.DMA,
            pltpu.SemaphoreType.DMA,
            pltpu.SemaphoreType.DMA,
        ],
    )
    def k(table_hbm, src2_hbm, dst2_hbm, ew2_hbm, out_hbm,
          sidx2, didx2, wv2, rows_a, rows_b, rows_c, rows_d, zbuf, acc,
          gsem_a, gsem_b, gsem_c, gsem_d,
          ssem_a, ssem_b, ssem_c, ssem_d, isem):
        cid = lax.axis_index("c")
        sid = lax.axis_index("s")
        wid = sid * _NC + cid
        crow = wid * _NCHUNK

        ld_s = pltpu.async_copy(src2_hbm.at[pl.ds(crow, _NCHUNK)], sidx2, isem)
        ld_d = pltpu.async_copy(dst2_hbm.at[pl.ds(crow, _NCHUNK)], didx2, isem)
        ld_w = pltpu.async_copy(ew2_hbm.at[pl.ds(crow, _NCHUNK)], wv2, isem)

        zero16 = jnp.zeros((16,), jnp.float32)

        @pl.loop(0, _ZB)
        def _zfill(i):
            zbuf[i, pl.ds(0, 16)] = zero16
            zbuf[i, pl.ds(16, 16)] = zero16

        @pl.loop(0, 5)
        def _zcopy(j):
            pltpu.sync_copy(zbuf, acc.at[pl.ds(sid * _RPS + j * _ZB, _ZB)])

        ld_s.wait()
        ld_d.wait()
        ld_w.wait()
        plsc.subcore_barrier()

        def gather(c, rows, sem):
            pltpu.async_copy(table_hbm.at[sidx2.at[c]], rows, sem)

        def wait_gather(c, rows, sem):
            pltpu.make_async_copy(table_hbm.at[sidx2.at[c]], rows, sem).wait()

        def mult(rows, c):
            @plsc.parallel_loop(0, _CH, step=16)
            def _grp(g):
                wgrp = wv2[c, pl.ds(g, 16)]
                for u in range(16):
                    w = wgrp[u]
                    rows[g + u, pl.ds(0, 16)] = rows[g + u, pl.ds(0, 16)] * w
                    rows[g + u, pl.ds(16, 16)] = rows[g + u, pl.ds(16, 16)] * w

        def scatter(c, rows, sem):
            pltpu.async_copy(rows, acc.at[didx2.at[c]], sem, add=True)

        def wait_scatter(c, rows, sem):
            pltpu.make_async_copy(rows, acc.at[didx2.at[c]], sem).wait()

        bufs = [(rows_a, gsem_a, ssem_a), (rows_b, gsem_b, ssem_b),
                (rows_c, gsem_c, ssem_c), (rows_d, gsem_d, ssem_d)]

        gather(0, rows_a, gsem_a)
        gather(1, rows_b, gsem_b)
        gather(2, rows_c, gsem_c)

        @pl.loop(0, _NQUAD)
        def _quad(t):
            c = 4 * t
            for j in range(4):
                bx, gx, sx = bufs[j]
                pv_rows, pv_gsem, pv_ssem = bufs[(j + 3) % 4]
                wait_gather(c + j, bx, gx)
                mult(bx, c + j)
                scatter(c + j, bx, sx)
                if j == 0:
                    @pl.when(t > 0)
                    def _(pv_rows=pv_rows, pv_ssem=pv_ssem, cw=c - 1):
                        wait_scatter(cw, pv_rows, pv_ssem)
                    gather(c + 3, pv_rows, pv_gsem)
                else:
                    wait_scatter(c + j - 1, pv_rows, pv_ssem)
                    nxt = c + j + 3

                    @pl.when(t < _NQUAD - 1)
                    def _(pv_rows=pv_rows, pv_gsem=pv_gsem, nxt=nxt):
                        gather(nxt, pv_rows, pv_gsem)

        wait_scatter(_NCHUNK - 1, rows_d, ssem_d)

        plsc.subcore_barrier()

        @pl.loop(0, 5)
        def _wb(j):
            r0 = sid * _RPS + j * _ZB
            pltpu.sync_copy(acc.at[pl.ds(r0, _ZB)],
                            out_hbm.at[pl.ds(cid * _NP + r0, _ZB)])

    return k(table, src2, dst2, ew2)


def _tc_proj0(xp, wr, wo, b):
    def body(x_ref, wr_ref, wo_ref, b_ref, t_ref, r_ref):
        xb = x_ref[...]
        t_ref[...] = jnp.dot(xb, wr_ref[...], preferred_element_type=jnp.float32)
        r_ref[...] = jnp.dot(xb, wo_ref[...], preferred_element_type=jnp.float32) + b_ref[...]

    return pl.pallas_call(
        body,
        grid=(_NB,),
        in_specs=[
            pl.BlockSpec((_BR, _DIN), lambda i: (i, 0)),
            pl.BlockSpec((_DIN, _DH), lambda i: (0, 0)),
            pl.BlockSpec((_DIN, _DH), lambda i: (0, 0)),
            pl.BlockSpec((1, _DH), lambda i: (0, 0)),
        ],
        out_specs=[
            pl.BlockSpec((_BR, _DH), lambda i: (i, 0)),
            pl.BlockSpec((_BR, _DH), lambda i: (i, 0)),
        ],
        out_shape=[jax.ShapeDtypeStruct((_NP, _DH), jnp.float32)] * 2,
    )(xp, wr, wo, b.reshape(1, _DH))


def _tc_mid(ap, r_prev, wr, wo, b):
    def body(a0_ref, a1_ref, rp_ref, wr_ref, wo_ref, b_ref, t_ref, r_ref):
        h = a0_ref[...] + a1_ref[...] + rp_ref[...]
        h = jnp.where(h > 0, h, 0.01 * h)
        t_ref[...] = jnp.dot(h, wr_ref[...], preferred_element_type=jnp.float32)
        r_ref[...] = jnp.dot(h, wo_ref[...], preferred_element_type=jnp.float32) + b_ref[...]

    return pl.pallas_call(
        body,
        grid=(_NB,),
        in_specs=[
            pl.BlockSpec((_BR, _DH), lambda i: (i, 0)),
            pl.BlockSpec((_BR, _DH), lambda i: (i + _NB, 0)),
            pl.BlockSpec((_BR, _DH), lambda i: (i, 0)),
            pl.BlockSpec((_DH, _DH), lambda i: (0, 0)),
            pl.BlockSpec((_DH, _DH), lambda i: (0, 0)),
            pl.BlockSpec((1, _DH), lambda i: (0, 0)),
        ],
        out_specs=[
            pl.BlockSpec((_BR, _DH), lambda i: (i, 0)),
            pl.BlockSpec((_BR, _DH), lambda i: (i, 0)),
        ],
        out_shape=[jax.ShapeDtypeStruct((_NP, _DH), jnp.float32)] * 2,
    )(ap, ap, r_prev, wr, wo, b.reshape(1, _DH))


def _tc_last_pre(ap, r_prev, wo, b):
    def body(a0_ref, a1_ref, rp_ref, wo_ref, b_ref, h_ref, r_ref):
        h = a0_ref[...] + a1_ref[...] + rp_ref[...]
        h = jnp.where(h > 0, h, 0.01 * h)
        h_ref[...] = h
        r_ref[...] = jnp.dot(h, wo_ref[...], preferred_element_type=jnp.float32) + b_ref[...]

    return pl.pallas_call(
        body,
        grid=(_NB,),
        in_specs=[
            pl.BlockSpec((_BR, _DH), lambda i: (i, 0)),
            pl.BlockSpec((_BR, _DH), lambda i: (i + _NB, 0)),
            pl.BlockSpec((_BR, _DH), lambda i: (i, 0)),
            pl.BlockSpec((_DH, _DOUT), lambda i: (0, 0)),
            pl.BlockSpec((1, _DOUT), lambda i: (0, 0)),
        ],
        out_specs=[
            pl.BlockSpec((_BR, _DH), lambda i: (i, 0)),
            pl.BlockSpec((_BR, _DOUT), lambda i: (i, 0)),
        ],
        out_shape=[
            jax.ShapeDtypeStruct((_NP, _DH), jnp.float32),
            jax.ShapeDtypeStruct((_NP, _DOUT), jnp.float32),
        ],
    )(ap, ap, r_prev, wo, b.reshape(1, _DOUT))


def _tc_final(ap, r2, wr):
    def body(a0_ref, a1_ref, r2_ref, wr_ref, o_ref):
        a = a0_ref[...] + a1_ref[...]
        o_ref[...] = jnp.dot(a, wr_ref[...], preferred_element_type=jnp.float32) + r2_ref[...]

    return pl.pallas_call(
        body,
        grid=(_NB,),
        in_specs=[
            pl.BlockSpec((_BR, _DH), lambda i: (i, 0)),
            pl.BlockSpec((_BR, _DH), lambda i: (i + _NB, 0)),
            pl.BlockSpec((_BR, _DOUT), lambda i: (i, 0)),
            pl.BlockSpec((_DH, _DOUT), lambda i: (0, 0)),
        ],
        out_specs=pl.BlockSpec((_BR, _DOUT), lambda i: (i, 0)),
        out_shape=jax.ShapeDtypeStruct((_NP, _DOUT), jnp.float32),
    )(ap, ap, r2, wr)


def kernel(x, edge_index, edge_weights,
           W_rel0, W_root0, b0,
           W_rel1, W_root1, b1,
           W_rel2, W_root2, b2):
    pad_e = _EP - _E
    pad_idx = (jnp.arange(pad_e, dtype=jnp.int32) * 8) % _NP
    src2 = jnp.concatenate(
        [edge_index[0].astype(jnp.int32), pad_idx]).reshape(_EP // _CH, _CH)
    dst2 = jnp.concatenate(
        [edge_index[1].astype(jnp.int32), pad_idx]).reshape(_EP // _CH, _CH)
    ew2 = jnp.pad(edge_weights.astype(jnp.float32), (0, pad_e)).reshape(_EP // _CH, _CH)
    xp = jnp.pad(x, ((0, _NP - _N), (0, 0)))

    def seg(table):
        return _seg_sum_sc(table, src2, dst2, ew2)

    t0, r0 = _tc_proj0(xp, W_rel0, W_root0, b0)
    a0 = seg(t0)
    t1, r1 = _tc_mid(a0, r0, W_rel1, W_root1, b1)
    a1 = seg(t1)
    h2, r2 = _tc_last_pre(a1, r1, W_root2, b2)
    a2 = seg(h2)
    return _tc_final(a2, r2, W_rel2)[:_N]


# speedup vs baseline: 2.1581x; 1.0070x over previous
import dataclasses
import functools

import jax
import jax.numpy as jnp
from jax import lax
from jax.experimental import pallas as pl
from jax.experimental.pallas import tpu as pltpu
from jax.experimental.pallas import tpu_sc as plsc

_N = 10000
_E = 320000
_DIN = 128
_DH = 32
_DOUT = 64

_NC = 2
_NS = 16
_NW = _NC * _NS
_CH = 128
_NCHUNK = 80
_EP = _NW * _NCHUNK * _CH
_NP = 10240
_RPS = _NP // _NS
_ZB = 128

_BR = 1280
_NB = 8


_NQUAD = _NCHUNK // 4


def _seg_sum_sc(table, src2, dst2, ew2):
    mesh = plsc.VectorSubcoreMesh(core_axis_name="c", subcore_axis_name="s")
    cp = pltpu.CompilerParams()
    if "needs_layout_passes" in pltpu.CompilerParams.__dataclass_fields__:
        cp = dataclasses.replace(cp, needs_layout_passes=False)
    if "use_tc_tiling_on_sc" in pltpu.CompilerParams.__dataclass_fields__:
        cp = dataclasses.replace(cp, use_tc_tiling_on_sc=False)

    @functools.partial(
        pl.kernel,
        compiler_params=cp,
        out_type=jax.ShapeDtypeStruct((_NC * _NP, _DH), jnp.float32),
        mesh=mesh,
        scratch_types=[
            pltpu.VMEM((_NCHUNK, _CH), jnp.int32),
            pltpu.VMEM((_NCHUNK, _CH), jnp.int32),
            pltpu.VMEM((_NCHUNK, _CH), jnp.float32),
            pltpu.VMEM((_CH, _DH), jnp.float32),
            pltpu.VMEM((_CH, _DH), jnp.float32),
            pltpu.VMEM((_CH, _DH), jnp.float32),
            pltpu.VMEM((_CH, _DH), jnp.float32),
            pltpu.VMEM((_ZB, _DH), jnp.float32),
            pltpu.VMEM_SHARED((_NP, _DH), jnp.float32),
            pltpu.SemaphoreType.DMA,
            pltpu.SemaphoreType.DMA,
            pltpu.SemaphoreType.DMA,
            pltpu.SemaphoreType.DMA,
            pltpu.SemaphoreType.DMA,
            pltpu.SemaphoreType.DMA,
            pltpu.SemaphoreType.DMA,
            pltpu.SemaphoreType.DMA,
            pltpu.SemaphoreType.DMA,
        ],
    )
    def k(table_hbm, src2_hbm, dst2_hbm, ew2_hbm, out_hbm,
          sidx2, didx2, wv2, rows_a, rows_b, rows_c, rows_d, zbuf, acc,
          gsem_a, gsem_b, gsem_c, gsem_d,
          ssem_a, ssem_b, ssem_c, ssem_d, isem):
        cid = lax.axis_index("c")
        sid = lax.axis_index("s")
        wid = sid * _NC + cid
        crow = wid * _NCHUNK

        ld_s = pltpu.async_copy(src2_hbm.at[pl.ds(crow, _NCHUNK)], sidx2, isem)
        ld_d = pltpu.async_copy(dst2_hbm.at[pl.ds(crow, _NCHUNK)], didx2, isem)
        ld_w = pltpu.async_copy(ew2_hbm.at[pl.ds(crow, _NCHUNK)], wv2, isem)

        zero16 = jnp.zeros((16,), jnp.float32)

        @pl.loop(0, _ZB)
        def _zfill(i):
            zbuf[i, pl.ds(0, 16)] = zero16
            zbuf[i, pl.ds(16, 16)] = zero16

        @pl.loop(0, 5)
        def _zcopy(j):
            pltpu.sync_copy(zbuf, acc.at[pl.ds(sid * _RPS + j * _ZB, _ZB)])

        def gather(c, rows, sem):
            pltpu.async_copy(table_hbm.at[sidx2.at[c]], rows, sem)

        def wait_gather(c, rows, sem):
            pltpu.make_async_copy(table_hbm.at[sidx2.at[c]], rows, sem).wait()

        def mult(rows, c):
            @plsc.parallel_loop(0, _CH, step=16)
            def _grp(g):
                wgrp = wv2[c, pl.ds(g, 16)]
                for u in range(16):
                    w = wgrp[u]
                    rows[g + u, pl.ds(0, 16)] = rows[g + u, pl.ds(0, 16)] * w
                    rows[g + u, pl.ds(16, 16)] = rows[g + u, pl.ds(16, 16)] * w

        def scatter(c, rows, sem):
            pltpu.async_copy(rows, acc.at[didx2.at[c]], sem, add=True)

        def wait_scatter(c, rows, sem):
            pltpu.make_async_copy(rows, acc.at[didx2.at[c]], sem).wait()

        bufs = [(rows_a, gsem_a, ssem_a), (rows_b, gsem_b, ssem_b),
                (rows_c, gsem_c, ssem_c), (rows_d, gsem_d, ssem_d)]

        ld_s.wait()
        gather(0, rows_a, gsem_a)
        gather(1, rows_b, gsem_b)
        gather(2, rows_c, gsem_c)
        ld_d.wait()
        ld_w.wait()
        plsc.subcore_barrier()

        @pl.loop(0, _NQUAD)
        def _quad(t):
            c = 4 * t
            for j in range(4):
                bx, gx, sx = bufs[j]
                pv_rows, pv_gsem, pv_ssem = bufs[(j + 3) % 4]
                wait_gather(c + j, bx, gx)
                mult(bx, c + j)
                scatter(c + j, bx, sx)
                if j == 0:
                    @pl.when(t > 0)
                    def _(pv_rows=pv_rows, pv_ssem=pv_ssem, cw=c - 1):
                        wait_scatter(cw, pv_rows, pv_ssem)
                    gather(c + 3, pv_rows, pv_gsem)
                else:
                    wait_scatter(c + j - 1, pv_rows, pv_ssem)
                    nxt = c + j + 3

                    @pl.when(t < _NQUAD - 1)
                    def _(pv_rows=pv_rows, pv_gsem=pv_gsem, nxt=nxt):
                        gather(nxt, pv_rows, pv_gsem)

        wait_scatter(_NCHUNK - 1, rows_d, ssem_d)

        plsc.subcore_barrier()

        @pl.loop(0, 5)
        def _wb(j):
            r0 = sid * _RPS + j * _ZB
            pltpu.sync_copy(acc.at[pl.ds(r0, _ZB)],
                            out_hbm.at[pl.ds(cid * _NP + r0, _ZB)])

    return k(table, src2, dst2, ew2)


def _tc_proj0(xp, wr, wo, b):
    def body(x_ref, wr_ref, wo_ref, b_ref, t_ref, r_ref):
        xb = x_ref[...]
        t_ref[...] = jnp.dot(xb, wr_ref[...], preferred_element_type=jnp.float32)
        r_ref[...] = jnp.dot(xb, wo_ref[...], preferred_element_type=jnp.float32) + b_ref[...]

    return pl.pallas_call(
        body,
        grid=(_NB,),
        in_specs=[
            pl.BlockSpec((_BR, _DIN), lambda i: (i, 0)),
            pl.BlockSpec((_DIN, _DH), lambda i: (0, 0)),
            pl.BlockSpec((_DIN, _DH), lambda i: (0, 0)),
            pl.BlockSpec((1, _DH), lambda i: (0, 0)),
        ],
        out_specs=[
            pl.BlockSpec((_BR, _DH), lambda i: (i, 0)),
            pl.BlockSpec((_BR, _DH), lambda i: (i, 0)),
        ],
        out_shape=[jax.ShapeDtypeStruct((_NP, _DH), jnp.float32)] * 2,
    )(xp, wr, wo, b.reshape(1, _DH))


def _tc_mid(ap, r_prev, wr, wo, b):
    def body(a0_ref, a1_ref, rp_ref, wr_ref, wo_ref, b_ref, t_ref, r_ref):
        h = a0_ref[...] + a1_ref[...] + rp_ref[...]
        h = jnp.where(h > 0, h, 0.01 * h)
        t_ref[...] = jnp.dot(h, wr_ref[...], preferred_element_type=jnp.float32)
        r_ref[...] = jnp.dot(h, wo_ref[...], preferred_element_type=jnp.float32) + b_ref[...]

    return pl.pallas_call(
        body,
        grid=(_NB,),
        in_specs=[
            pl.BlockSpec((_BR, _DH), lambda i: (i, 0)),
            pl.BlockSpec((_BR, _DH), lambda i: (i + _NB, 0)),
            pl.BlockSpec((_BR, _DH), lambda i: (i, 0)),
            pl.BlockSpec((_DH, _DH), lambda i: (0, 0)),
            pl.BlockSpec((_DH, _DH), lambda i: (0, 0)),
            pl.BlockSpec((1, _DH), lambda i: (0, 0)),
        ],
        out_specs=[
            pl.BlockSpec((_BR, _DH), lambda i: (i, 0)),
            pl.BlockSpec((_BR, _DH), lambda i: (i, 0)),
        ],
        out_shape=[jax.ShapeDtypeStruct((_NP, _DH), jnp.float32)] * 2,
    )(ap, ap, r_prev, wr, wo, b.reshape(1, _DH))


def _tc_last_pre(ap, r_prev, wo, b):
    def body(a0_ref, a1_ref, rp_ref, wo_ref, b_ref, h_ref, r_ref):
        h = a0_ref[...] + a1_ref[...] + rp_ref[...]
        h = jnp.where(h > 0, h, 0.01 * h)
        h_ref[...] = h
        r_ref[...] = jnp.dot(h, wo_ref[...], preferred_element_type=jnp.float32) + b_ref[...]

    return pl.pallas_call(
        body,
        grid=(_NB,),
        in_specs=[
            pl.BlockSpec((_BR, _DH), lambda i: (i, 0)),
            pl.BlockSpec((_BR, _DH), lambda i: (i + _NB, 0)),
            pl.BlockSpec((_BR, _DH), lambda i: (i, 0)),
            pl.BlockSpec((_DH, _DOUT), lambda i: (0, 0)),
            pl.BlockSpec((1, _DOUT), lambda i: (0, 0)),
        ],
        out_specs=[
            pl.BlockSpec((_BR, _DH), lambda i: (i, 0)),
            pl.BlockSpec((_BR, _DOUT), lambda i: (i, 0)),
        ],
        out_shape=[
            jax.ShapeDtypeStruct((_NP, _DH), jnp.float32),
            jax.ShapeDtypeStruct((_NP, _DOUT), jnp.float32),
        ],
    )(ap, ap, r_prev, wo, b.reshape(1, _DOUT))


def _tc_final(ap, r2, wr):
    def body(a0_ref, a1_ref, r2_ref, wr_ref, o_ref):
        a = a0_ref[...] + a1_ref[...]
        o_ref[...] = jnp.dot(a, wr_ref[...], preferred_element_type=jnp.float32) + r2_ref[...]

    return pl.pallas_call(
        body,
        grid=(_NB,),
        in_specs=[
            pl.BlockSpec((_BR, _DH), lambda i: (i, 0)),
            pl.BlockSpec((_BR, _DH), lambda i: (i + _NB, 0)),
            pl.BlockSpec((_BR, _DOUT), lambda i: (i, 0)),
            pl.BlockSpec((_DH, _DOUT), lambda i: (0, 0)),
        ],
        out_specs=pl.BlockSpec((_BR, _DOUT), lambda i: (i, 0)),
        out_shape=jax.ShapeDtypeStruct((_NP, _DOUT), jnp.float32),
    )(ap, ap, r2, wr)


def kernel(x, edge_index, edge_weights,
           W_rel0, W_root0, b0,
           W_rel1, W_root1, b1,
           W_rel2, W_root2, b2):
    pad_e = _EP - _E
    pad_idx = (jnp.arange(pad_e, dtype=jnp.int32) * 8) % _NP
    src2 = jnp.concatenate(
        [edge_index[0].astype(jnp.int32), pad_idx]).reshape(_EP // _CH, _CH)
    dst2 = jnp.concatenate(
        [edge_index[1].astype(jnp.int32), pad_idx]).reshape(_EP // _CH, _CH)
    ew2 = jnp.pad(edge_weights.astype(jnp.float32), (0, pad_e)).reshape(_EP // _CH, _CH)
    xp = jnp.pad(x, ((0, _NP - _N), (0, 0)))

    def seg(table):
        return _seg_sum_sc(table, src2, dst2, ew2)

    t0, r0 = _tc_proj0(xp, W_rel0, W_root0, b0)
    a0 = seg(t0)
    t1, r1 = _tc_mid(a0, r0, W_rel1, W_root1, b1)
    a1 = seg(t1)
    h2, r2 = _tc_last_pre(a1, r1, W_root2, b2)
    a2 = seg(h2)
    return _tc_final(a2, r2, W_rel2)[:_N]
